# Initial kernel scaffold; baseline (speedup 1.0000x reference)
import jax
import jax.numpy as jnp
from jax.experimental import pallas as pl


def kernel(x):
    raise NotImplementedError("write your pallas kernel here")


# speedup vs baseline: 32.6776x; 32.6776x over previous
import functools

import jax
import jax.numpy as jnp
from jax.experimental import pallas as pl
from jax.experimental.pallas import tpu as pltpu

_BISECT_ITERS = 16
_NEWTON_ITERS = 2
_COL_BLOCK = 256


def _sparsemax_body(x_ref, o_ref):
    x = x_ref[...]
    m = jnp.max(x, axis=0, keepdims=True)
    lo = m - 1.0
    hi = m

    def bisect(_, carry):
        lo, hi = carry
        mid = 0.5 * (lo + hi)
        s = jnp.sum(jnp.maximum(x - mid, 0.0), axis=0, keepdims=True)
        go_right = s >= 1.0
        return jnp.where(go_right, mid, lo), jnp.where(go_right, hi, mid)

    lo, hi = jax.lax.fori_loop(0, _BISECT_ITERS, bisect, (lo, hi))
    tau = lo

    def newton(_, tau):
        mask = x > tau
        k = jnp.sum(mask.astype(jnp.float32), axis=0, keepdims=True)
        s = jnp.sum(jnp.where(mask, x, 0.0), axis=0, keepdims=True)
        return (s - 1.0) / k

    tau = jax.lax.fori_loop(0, _NEWTON_ITERS, newton, tau)
    o_ref[...] = jnp.maximum(x - tau, 0.0)


@jax.jit
def kernel(x):
    v, n = x.shape
    grid = (n // _COL_BLOCK,)
    return pl.pallas_call(
        _sparsemax_body,
        grid=grid,
        in_specs=[pl.BlockSpec((v, _COL_BLOCK), lambda j: (0, j))],
        out_specs=pl.BlockSpec((v, _COL_BLOCK), lambda j: (0, j)),
        out_shape=jax.ShapeDtypeStruct((v, n), x.dtype),
        compiler_params=pltpu.CompilerParams(
            dimension_semantics=("arbitrary",),
        ),
    )(x)


# speedup vs baseline: 43.5076x; 1.3314x over previous
import functools

import jax
import jax.numpy as jnp
from jax.experimental import pallas as pl
from jax.experimental.pallas import tpu as pltpu

_BISECT_ITERS = 10
_NEWTON_ITERS = 2
_COL_BLOCK = 256


def _sparsemax_body(x_ref, o_ref):
    x = x_ref[...]
    n = jnp.float32(x.shape[0])
    m = jnp.max(x, axis=0, keepdims=True)
    lo = m - 1.0
    hi = m

    def bisect(_, carry):
        lo, hi = carry
        mid = 0.5 * (lo + hi)
        s = jnp.sum(jnp.maximum(x - mid, 0.0), axis=0, keepdims=True)
        go_right = s >= 1.0
        return jnp.where(go_right, mid, lo), jnp.where(go_right, hi, mid)

    lo, hi = jax.lax.fori_loop(0, _BISECT_ITERS, bisect, (lo, hi))
    tau = lo

    def newton(_, tau):
        r = jnp.maximum(x - tau, 0.0)
        f = jnp.sum(r, axis=0, keepdims=True) - 1.0
        k = jnp.sum((r > 0.0).astype(jnp.float32), axis=0, keepdims=True)
        return tau + f / k

    tau = jax.lax.fori_loop(0, _NEWTON_ITERS, newton, tau)
    o_ref[...] = jnp.maximum(x - tau, 0.0)


@jax.jit
def kernel(x):
    v, n = x.shape
    grid = (n // _COL_BLOCK,)
    return pl.pallas_call(
        _sparsemax_body,
        grid=grid,
        in_specs=[pl.BlockSpec((v, _COL_BLOCK), lambda j: (0, j))],
        out_specs=pl.BlockSpec((v, _COL_BLOCK), lambda j: (0, j)),
        out_shape=jax.ShapeDtypeStruct((v, n), x.dtype),
        compiler_params=pltpu.CompilerParams(
            dimension_semantics=("arbitrary",),
        ),
    )(x)


# speedup vs baseline: 58.8707x; 1.3531x over previous
import functools

import jax
import jax.numpy as jnp
from jax.experimental import pallas as pl
from jax.experimental.pallas import tpu as pltpu

_BISECT_ITERS = 10
_NEWTON_ITERS = 2
_COL_BLOCK = 256


def _sparsemax_body(x_ref, o_ref):
    x = x_ref[...]
    v = x.shape[0]
    ones = jnp.ones((1, v), dtype=jnp.float32)

    def colsum(a):
        return jax.lax.dot_general(
            ones, a, (((1,), (0,)), ((), ())),
            preferred_element_type=jnp.float32)

    m = jnp.max(x, axis=0, keepdims=True)
    lo = m - 1.0
    hi = m

    def bisect(_, carry):
        lo, hi = carry
        mid = 0.5 * (lo + hi)
        s = colsum(jnp.maximum(x - mid, 0.0))
        go_right = s >= 1.0
        return jnp.where(go_right, mid, lo), jnp.where(go_right, hi, mid)

    lo, hi = jax.lax.fori_loop(0, _BISECT_ITERS, bisect, (lo, hi))
    tau = lo

    def newton(_, tau):
        r = jnp.maximum(x - tau, 0.0)
        f = colsum(r) - 1.0
        k = colsum(jnp.where(r > 0.0, 1.0, 0.0))
        return tau + f / k

    tau = jax.lax.fori_loop(0, _NEWTON_ITERS, newton, tau)
    o_ref[...] = jnp.maximum(x - tau, 0.0)


@jax.jit
def kernel(x):
    v, n = x.shape
    grid = (n // _COL_BLOCK,)
    return pl.pallas_call(
        _sparsemax_body,
        grid=grid,
        in_specs=[pl.BlockSpec((v, _COL_BLOCK), lambda j: (0, j))],
        out_specs=pl.BlockSpec((v, _COL_BLOCK), lambda j: (0, j)),
        out_shape=jax.ShapeDtypeStruct((v, n), x.dtype),
        compiler_params=pltpu.CompilerParams(
            dimension_semantics=("arbitrary",),
        ),
    )(x)
